# Initial kernel scaffold; baseline (speedup 1.0000x reference)
import jax
import jax.numpy as jnp
from jax.experimental import pallas as pl


def kernel(input):
    raise NotImplementedError("write your pallas kernel here")


# speedup vs baseline: 8.1451x; 8.1451x over previous
import functools

import jax
import jax.numpy as jnp
from jax import lax
from jax.experimental import pallas as pl
from jax.experimental.pallas import tpu as pltpu
from jax.experimental.pallas import tpu_sc as plsc

R = 128
N = 32768
L = 16
NV = N // L
K = 2048
WPL = K // L
NC = 2
NS = 16
NW = NC * NS
RPW = R // NW

_mesh = plsc.VectorSubcoreMesh(
    core_axis_name="c", subcore_axis_name="s", num_cores=NC, num_subcores=NS
)


@functools.partial(
    pl.kernel,
    out_type=jax.ShapeDtypeStruct((R, N), jnp.float32),
    mesh=_mesh,
    compiler_params=pltpu.CompilerParams(needs_layout_passes=False),
    scratch_types=[
        pltpu.VMEM((N,), jnp.float32),
        pltpu.VMEM((K,), jnp.float32),
        pltpu.VMEM((WPL, L), jnp.float32),
        pltpu.VMEM((WPL, L), jnp.float32),
    ],
)
def _sparsemax_sc(x_hbm, out_hbm, row_v, cnt_v, csuf_v, ssuf_v):
    wid = lax.axis_index("s") * NC + lax.axis_index("c")

    zeros = jnp.zeros((L,), jnp.float32)
    ones = jnp.ones((L,), jnp.float32)
    lane_i = lax.iota(jnp.int32, L)
    lane_f = lane_i.astype(jnp.float32)
    inv_k = jnp.float32(1.0 / K)

    def zero_body(w, _):
        cnt_v[pl.ds(w * L, L)] = zeros
        return 0
    lax.fori_loop(0, WPL, zero_body, 0)

    for j in range(RPW):
        ridx = wid * RPW + j
        pltpu.sync_copy(x_hbm.at[ridx], row_v)

        def max_body(i, acc):
            return jnp.maximum(acc, row_v[pl.ds(i * L, L)])
        m_vec = lax.fori_loop(0, NV, max_body, jnp.full((L,), -jnp.inf, jnp.float32))
        m_s = jnp.max(m_vec)
        lo_s = m_s - jnp.float32(1.0)
        lo_v = jnp.full((L,), lo_s, jnp.float32)

        kf_v = jnp.full((L,), jnp.float32(K), jnp.float32)
        def hist_body(i, _):
            x = row_v[pl.ds(i * L, L)]
            t = (x - lo_v) * kf_v
            idx = jnp.clip(t.astype(jnp.int32), 0, K - 1)
            sidx = ((idx & (WPL - 1)) << 4) | (idx >> 7)
            plsc.addupdate_scatter(cnt_v, [sidx], ones, mask=x >= lo_v)
            return 0
        lax.fori_loop(0, NV, hist_body, 0)

        lane_base = lo_v + lane_f * jnp.float32(1.0 / L)
        def scan1_body(wr, carry):
            c_run, s_run = carry
            w = WPL - 1 - wr
            v = cnt_v[pl.ds(w * L, L)]
            mid = lane_base + (w + 0.5) * inv_k
            c_run = c_run + v
            s_run = s_run + v * mid
            csuf_v[w] = c_run
            ssuf_v[w] = s_run
            return c_run, s_run
        c_tot, s_tot = lax.fori_loop(0, WPL, scan1_body, (zeros, zeros))

        tc_excl = lax.rev(plsc.cumsum(lax.rev(c_tot, (0,))), (0,)) - c_tot
        ts_excl = lax.rev(plsc.cumsum(lax.rev(s_tot, (0,))), (0,)) - s_tot

        def scan2_body(w, rmax):
            cs = csuf_v[w] + tc_excl
            ss = ssuf_v[w] + ts_excl
            edge = lane_base + w * inv_k
            g = ss - edge * cs - jnp.float32(1.0)
            r = (ss - jnp.float32(1.0)) / jnp.maximum(cs, jnp.float32(1e-30))
            rmax = jnp.maximum(rmax, jnp.where(g >= 0, r, jnp.float32(-3.0)))
            cnt_v[pl.ds(w * L, L)] = zeros
            return rmax
        rmax = lax.fori_loop(0, WPL, scan2_body, jnp.full((L,), -3.0, jnp.float32))
        tau0 = jnp.clip(jnp.max(rmax), lo_s, m_s - jnp.float32(1.0 / N))
        tau0_v = jnp.full((L,), tau0, jnp.float32)

        def stat_body(i, carry):
            s_acc, k_acc = carry
            d = row_v[pl.ds(i * L, L)] - tau0_v
            s_acc = s_acc + jnp.maximum(d, zeros)
            k_acc = k_acc + jnp.where(d > 0, ones, zeros)
            return s_acc, k_acc
        s_acc, k_acc = lax.fori_loop(0, NV, stat_body, (zeros, zeros))
        sp_v = jnp.full((L,), jnp.sum(s_acc), jnp.float32)
        kp_v = jnp.maximum(jnp.full((L,), jnp.sum(k_acc), jnp.float32), ones)
        tau1_v = tau0_v + (sp_v - ones) / kp_v

        def out_body(i, _):
            sl = pl.ds(i * L, L)
            row_v[sl] = jnp.maximum(row_v[sl] - tau1_v, zeros)
            return 0
        lax.fori_loop(0, NV, out_body, 0)
        pltpu.sync_copy(row_v, out_hbm.at[ridx])


def kernel(input):
    return _sparsemax_sc(input)


# speedup vs baseline: 12.8095x; 1.5727x over previous
import functools

import jax
import jax.numpy as jnp
from jax import lax
from jax.experimental import pallas as pl
from jax.experimental.pallas import tpu as pltpu
from jax.experimental.pallas import tpu_sc as plsc

R = 128
N = 32768
L = 16
NV = N // L
K = 2048
WPL = K // L
NC = 2
NS = 16
NW = NC * NS
RPW = R // NW

_mesh = plsc.VectorSubcoreMesh(
    core_axis_name="c", subcore_axis_name="s", num_cores=NC, num_subcores=NS
)


@functools.partial(
    pl.kernel,
    out_type=jax.ShapeDtypeStruct((R, N), jnp.float32),
    mesh=_mesh,
    compiler_params=pltpu.CompilerParams(needs_layout_passes=False),
    scratch_types=[
        pltpu.VMEM((N,), jnp.float32),
        pltpu.VMEM((K,), jnp.float32),
        pltpu.VMEM((WPL, L), jnp.float32),
        pltpu.VMEM((WPL, L), jnp.float32),
    ],
)
def _sparsemax_sc(x_hbm, out_hbm, row_v, cnt_v, csuf_v, ssuf_v):
    wid = lax.axis_index("s") * NC + lax.axis_index("c")

    zeros = jnp.zeros((L,), jnp.float32)
    ones = jnp.ones((L,), jnp.float32)
    lane_i = lax.iota(jnp.int32, L)
    lane_f = lane_i.astype(jnp.float32)
    inv_k = jnp.float32(1.0 / K)

    def zero_body(w, _):
        cnt_v[pl.ds(w * L, L)] = zeros
        return 0
    lax.fori_loop(0, WPL, zero_body, 0, unroll=4)

    for j in range(RPW):
        ridx = wid * RPW + j
        pltpu.sync_copy(x_hbm.at[ridx], row_v)

        def max_body(i, acc):
            return jnp.maximum(acc, row_v[pl.ds(i * L, L)])
        m_vec = lax.fori_loop(0, NV, max_body, jnp.full((L,), -jnp.inf, jnp.float32), unroll=8)
        m_s = jnp.max(m_vec)
        lo_s = m_s - jnp.float32(1.0)
        lo_v = jnp.full((L,), lo_s, jnp.float32)

        kf_v = jnp.full((L,), jnp.float32(K), jnp.float32)
        def hist_body(i, _):
            x = row_v[pl.ds(i * L, L)]
            t = (x - lo_v) * kf_v
            idx = jnp.clip(t.astype(jnp.int32), 0, K - 1)
            sidx = ((idx & (WPL - 1)) << 4) | (idx >> 7)
            plsc.addupdate_scatter(cnt_v, [sidx], ones, mask=x >= lo_v)
            return 0
        lax.fori_loop(0, NV, hist_body, 0, unroll=8)

        lane_base = lo_v + lane_f * jnp.float32(1.0 / L)
        def scan1_body(wr, carry):
            c_run, s_run = carry
            w = WPL - 1 - wr
            v = cnt_v[pl.ds(w * L, L)]
            mid = lane_base + (w + 0.5) * inv_k
            c_run = c_run + v
            s_run = s_run + v * mid
            csuf_v[w] = c_run
            ssuf_v[w] = s_run
            return c_run, s_run
        c_tot, s_tot = lax.fori_loop(0, WPL, scan1_body, (zeros, zeros), unroll=4)

        tc_excl = lax.rev(plsc.cumsum(lax.rev(c_tot, (0,))), (0,)) - c_tot
        ts_excl = lax.rev(plsc.cumsum(lax.rev(s_tot, (0,))), (0,)) - s_tot

        def scan2_body(w, rmax):
            cs = csuf_v[w] + tc_excl
            ss = ssuf_v[w] + ts_excl
            edge = lane_base + w * inv_k
            g = ss - edge * cs - jnp.float32(1.0)
            r = (ss - jnp.float32(1.0)) / jnp.maximum(cs, jnp.float32(1e-30))
            rmax = jnp.maximum(rmax, jnp.where(g >= 0, r, jnp.float32(-3.0)))
            cnt_v[pl.ds(w * L, L)] = zeros
            return rmax
        rmax = lax.fori_loop(0, WPL, scan2_body, jnp.full((L,), -3.0, jnp.float32), unroll=4)
        tau0 = jnp.clip(jnp.max(rmax), lo_s, m_s - jnp.float32(1.0 / N))
        tau0_v = jnp.full((L,), tau0, jnp.float32)

        def stat_body(i, carry):
            s_acc, k_acc = carry
            d = row_v[pl.ds(i * L, L)] - tau0_v
            s_acc = s_acc + jnp.maximum(d, zeros)
            k_acc = k_acc + jnp.where(d > 0, ones, zeros)
            return s_acc, k_acc
        s_acc, k_acc = lax.fori_loop(0, NV, stat_body, (zeros, zeros), unroll=8)
        sp_v = jnp.full((L,), jnp.sum(s_acc), jnp.float32)
        kp_v = jnp.maximum(jnp.full((L,), jnp.sum(k_acc), jnp.float32), ones)
        tau1_v = tau0_v + (sp_v - ones) / kp_v

        def out_body(i, _):
            sl = pl.ds(i * L, L)
            row_v[sl] = jnp.maximum(row_v[sl] - tau1_v, zeros)
            return 0
        lax.fori_loop(0, NV, out_body, 0, unroll=8)
        pltpu.sync_copy(row_v, out_hbm.at[ridx])


def kernel(input):
    return _sparsemax_sc(input)


# speedup vs baseline: 27.7708x; 2.1680x over previous
import functools

import jax
import jax.numpy as jnp
from jax import lax
from jax.experimental import pallas as pl
from jax.experimental.pallas import tpu as pltpu
from jax.experimental.pallas import tpu_sc as plsc

R = 128
N = 32768
L = 16
NV = N // L
K = 2048
WPL = K // L
NC = 2
NS = 16
NW = NC * NS
RPW = R // NW
ACC = 4

_mesh = plsc.VectorSubcoreMesh(
    core_axis_name="c", subcore_axis_name="s", num_cores=NC, num_subcores=NS
)


@functools.partial(
    pl.kernel,
    out_type=jax.ShapeDtypeStruct((R, N), jnp.float32),
    mesh=_mesh,
    compiler_params=pltpu.CompilerParams(needs_layout_passes=False),
    scratch_types=[
        pltpu.VMEM((N,), jnp.float32),
        pltpu.VMEM((K,), jnp.float32),
        pltpu.VMEM((WPL, L), jnp.float32),
        pltpu.VMEM((WPL, L), jnp.float32),
    ],
)
def _sparsemax_sc(x_hbm, out_hbm, row_v, cnt_v, csuf_v, ssuf_v):
    wid = lax.axis_index("s") * NC + lax.axis_index("c")

    zeros = jnp.zeros((L,), jnp.float32)
    ones = jnp.ones((L,), jnp.float32)
    lane_i = lax.iota(jnp.int32, L)
    lane_f = lane_i.astype(jnp.float32)
    inv_k = jnp.float32(1.0 / K)

    @plsc.parallel_loop(0, WPL, unroll=4)
    def _(w):
        cnt_v[pl.ds(w * L, L)] = zeros

    for j in range(RPW):
        ridx = wid * RPW + j
        pltpu.sync_copy(x_hbm.at[ridx], row_v)

        @plsc.parallel_loop(0, NV, ACC, unroll=2,
                            carry=tuple(jnp.full((L,), -jnp.inf, jnp.float32)
                                        for _ in range(ACC)))
        def m_accs(i, accs):
            return tuple(
                jnp.maximum(a, row_v[pl.ds((i + u) * L, L)])
                for u, a in enumerate(accs)
            )
        m_vec = functools.reduce(jnp.maximum, m_accs)
        m_s = jnp.max(m_vec)
        lo_s = m_s - jnp.float32(1.0)
        lo_v = jnp.full((L,), lo_s, jnp.float32)

        kf_v = jnp.full((L,), jnp.float32(K), jnp.float32)

        @plsc.parallel_loop(0, NV, unroll=8)
        def _(i):
            x = row_v[pl.ds(i * L, L)]
            t = (x - lo_v) * kf_v
            idx = jnp.clip(t.astype(jnp.int32), 0, K - 1)
            sidx = ((idx & (WPL - 1)) << 4) | (idx >> 7)
            plsc.addupdate_scatter(cnt_v, [sidx], ones, mask=x >= lo_v)

        lane_base = lo_v + lane_f * jnp.float32(1.0 / L)

        def scan1_body(wr, carry):
            c_run, s_run = carry
            w = WPL - 1 - wr
            v = cnt_v[pl.ds(w * L, L)]
            mid = lane_base + (w + 0.5) * inv_k
            c_run = c_run + v
            s_run = s_run + v * mid
            csuf_v[w] = c_run
            ssuf_v[w] = s_run
            return c_run, s_run
        c_tot, s_tot = lax.fori_loop(0, WPL, scan1_body, (zeros, zeros), unroll=4)

        tc_excl = lax.rev(plsc.cumsum(lax.rev(c_tot, (0,))), (0,)) - c_tot
        ts_excl = lax.rev(plsc.cumsum(lax.rev(s_tot, (0,))), (0,)) - s_tot

        @plsc.parallel_loop(0, WPL, unroll=4, carry=jnp.full((L,), -3.0, jnp.float32))
        def rmax(w, acc):
            cs = csuf_v[w] + tc_excl
            ss = ssuf_v[w] + ts_excl
            edge = lane_base + w * inv_k
            g = ss - edge * cs - jnp.float32(1.0)
            r = (ss - jnp.float32(1.0)) / jnp.maximum(cs, jnp.float32(1e-30))
            cnt_v[pl.ds(w * L, L)] = zeros
            return jnp.maximum(acc, jnp.where(g >= 0, r, jnp.float32(-3.0)))
        tau0 = jnp.clip(jnp.max(rmax), lo_s, m_s - jnp.float32(1.0 / N))
        tau0_v = jnp.full((L,), tau0, jnp.float32)

        @plsc.parallel_loop(0, NV, ACC, unroll=2,
                            carry=tuple(zeros for _ in range(2 * ACC)))
        def sk_accs(i, accs):
            out = []
            for u in range(ACC):
                d = row_v[pl.ds((i + u) * L, L)] - tau0_v
                out.append(accs[u] + jnp.maximum(d, zeros))
                out.append(accs[ACC + u] + jnp.where(d > 0, ones, zeros))
            return tuple(out[0::2]) + tuple(out[1::2])
        s_vec = functools.reduce(jnp.add, sk_accs[:ACC])
        k_vec = functools.reduce(jnp.add, sk_accs[ACC:])
        sp_v = jnp.full((L,), jnp.sum(s_vec), jnp.float32)
        kp_v = jnp.maximum(jnp.full((L,), jnp.sum(k_vec), jnp.float32), ones)
        tau1_v = tau0_v + (sp_v - ones) / kp_v

        @plsc.parallel_loop(0, NV, unroll=8)
        def _(i):
            sl = pl.ds(i * L, L)
            row_v[sl] = jnp.maximum(row_v[sl] - tau1_v, zeros)

        pltpu.sync_copy(row_v, out_hbm.at[ridx])


def kernel(input):
    return _sparsemax_sc(input)


# speedup vs baseline: 32.9056x; 1.1849x over previous
import functools

import jax
import jax.numpy as jnp
from jax import lax
from jax.experimental import pallas as pl
from jax.experimental.pallas import tpu as pltpu
from jax.experimental.pallas import tpu_sc as plsc

R = 128
N = 32768
L = 16
NV = N // L
K = 2048
WPL = K // L
NC = 2
NS = 16
NW = NC * NS
RPW = R // NW
ACC = 4
NBUF = 3

_mesh = plsc.VectorSubcoreMesh(
    core_axis_name="c", subcore_axis_name="s", num_cores=NC, num_subcores=NS
)


@functools.partial(
    pl.kernel,
    out_type=jax.ShapeDtypeStruct((R, N), jnp.float32),
    mesh=_mesh,
    compiler_params=pltpu.CompilerParams(needs_layout_passes=False),
    scratch_types=[
        pltpu.VMEM((N,), jnp.float32),
        pltpu.VMEM((N,), jnp.float32),
        pltpu.VMEM((N,), jnp.float32),
        pltpu.VMEM((K,), jnp.float32),
        pltpu.SemaphoreType.DMA,
        pltpu.SemaphoreType.DMA,
        pltpu.SemaphoreType.DMA,
        pltpu.SemaphoreType.DMA,
        pltpu.SemaphoreType.DMA,
        pltpu.SemaphoreType.DMA,
    ],
)
def _sparsemax_sc(x_hbm, out_hbm, b0, b1, b2, cnt_v,
                  is0, is1, is2, os0, os1, os2):
    wid = lax.axis_index("s") * NC + lax.axis_index("c")
    bufs = (b0, b1, b2)
    isems = (is0, is1, is2)
    osems = (os0, os1, os2)

    zeros = jnp.zeros((L,), jnp.float32)
    ones = jnp.ones((L,), jnp.float32)
    lane_i = lax.iota(jnp.int32, L)
    lane_f = lane_i.astype(jnp.float32)
    inv_k = jnp.float32(1.0 / K)

    @plsc.parallel_loop(0, WPL, unroll=4)
    def _(w):
        cnt_v[pl.ds(w * L, L)] = zeros

    row0 = wid * RPW
    in_h = {}
    out_h = {}
    for j in range(min(NBUF, RPW)):
        in_h[j] = pltpu.async_copy(x_hbm.at[row0 + j], bufs[j], isems[j])

    for j in range(RPW):
        b = j % NBUF
        row_v = bufs[b]
        if j >= NBUF - 1 and j + 1 < RPW:
            out_h[j + 1 - NBUF].wait()
            in_h[j + 1] = pltpu.async_copy(
                x_hbm.at[row0 + j + 1], bufs[(j + 1) % NBUF], isems[(j + 1) % NBUF]
            )
        in_h[j].wait()

        @plsc.parallel_loop(0, NV, ACC, unroll=2,
                            carry=tuple(jnp.full((L,), -jnp.inf, jnp.float32)
                                        for _ in range(ACC)))
        def m_accs(i, accs):
            return tuple(
                jnp.maximum(a, row_v[pl.ds((i + u) * L, L)])
                for u, a in enumerate(accs)
            )
        m_vec = functools.reduce(jnp.maximum, m_accs)
        m_s = jnp.max(m_vec)
        lo_s = m_s - jnp.float32(1.0)
        lo_v = jnp.full((L,), lo_s, jnp.float32)

        kf_v = jnp.full((L,), jnp.float32(K), jnp.float32)

        @plsc.parallel_loop(0, NV, unroll=8)
        def _(i):
            x = row_v[pl.ds(i * L, L)]
            t = (x - lo_v) * kf_v
            idx = jnp.minimum(t.astype(jnp.int32), K - 1)
            sidx = ((idx & (WPL - 1)) << 4) | (idx >> 7)
            plsc.addupdate_scatter(cnt_v, [sidx], ones, mask=x >= lo_v)

        lane_base = lo_v + lane_f * jnp.float32(1.0 / L)

        @plsc.parallel_loop(0, WPL, unroll=4, carry=(zeros, zeros))
        def tot(w, carry):
            c_run, s_run = carry
            v = cnt_v[pl.ds(w * L, L)]
            mid = lane_base + (w + 0.5) * inv_k
            return c_run + v, s_run + v * mid
        c_tot, s_tot = tot

        tc_excl = lax.rev(plsc.cumsum(lax.rev(c_tot, (0,))), (0,)) - c_tot
        ts_excl = lax.rev(plsc.cumsum(lax.rev(s_tot, (0,))), (0,)) - s_tot

        @plsc.parallel_loop(0, WPL, unroll=4,
                            carry=(tc_excl, ts_excl,
                                   jnp.full((L,), -3.0, jnp.float32)))
        def scan2(wr, carry):
            c_run, s_run, acc = carry
            w = WPL - 1 - wr
            v = cnt_v[pl.ds(w * L, L)]
            mid = lane_base + (w + 0.5) * inv_k
            cs = c_run + v
            ss = s_run + v * mid
            edge = lane_base + w * inv_k
            g = ss - edge * cs - jnp.float32(1.0)
            r = (ss - jnp.float32(1.0)) / jnp.maximum(cs, jnp.float32(1e-30))
            cnt_v[pl.ds(w * L, L)] = zeros
            return cs, ss, jnp.maximum(acc, jnp.where(g >= 0, r, jnp.float32(-3.0)))
        rmax = scan2[2]
        tau0 = jnp.clip(jnp.max(rmax), lo_s, m_s - jnp.float32(1.0 / N))
        tau0_v = jnp.full((L,), tau0, jnp.float32)

        @plsc.parallel_loop(0, NV, ACC, unroll=2,
                            carry=tuple(zeros for _ in range(2 * ACC)))
        def sk_accs(i, accs):
            s_out = []
            k_out = []
            for u in range(ACC):
                d = row_v[pl.ds((i + u) * L, L)] - tau0_v
                s_out.append(accs[u] + jnp.maximum(d, zeros))
                k_out.append(accs[ACC + u] + jnp.where(d > 0, ones, zeros))
            return tuple(s_out) + tuple(k_out)
        s_vec = functools.reduce(jnp.add, sk_accs[:ACC])
        k_vec = functools.reduce(jnp.add, sk_accs[ACC:])
        sp_v = jnp.full((L,), jnp.sum(s_vec), jnp.float32)
        kp_v = jnp.maximum(jnp.full((L,), jnp.sum(k_vec), jnp.float32), ones)
        tau1_v = tau0_v + (sp_v - ones) / kp_v

        @plsc.parallel_loop(0, NV, unroll=8)
        def _(i):
            sl = pl.ds(i * L, L)
            row_v[sl] = jnp.maximum(row_v[sl] - tau1_v, zeros)

        out_h[j] = pltpu.async_copy(row_v, out_hbm.at[row0 + j], osems[b])

    waited = {j + 1 - NBUF for j in range(RPW) if j >= NBUF - 1 and j + 1 < RPW}
    for j in range(RPW):
        if j not in waited:
            out_h[j].wait()


def kernel(input):
    return _sparsemax_sc(input)


# speedup vs baseline: 34.6550x; 1.0532x over previous
import functools

import jax
import jax.numpy as jnp
from jax import lax
from jax.experimental import pallas as pl
from jax.experimental.pallas import tpu as pltpu
from jax.experimental.pallas import tpu_sc as plsc

R = 128
N = 32768
L = 16
NV = N // L
K = 2048
KS = K * (1.0 - 2.0 ** -12)
WPL = K // L
NC = 2
NS = 16
NW = NC * NS
RPW = R // NW
ACC = 4
NBUF = 3

_mesh = plsc.VectorSubcoreMesh(
    core_axis_name="c", subcore_axis_name="s", num_cores=NC, num_subcores=NS
)


@functools.partial(
    pl.kernel,
    out_type=jax.ShapeDtypeStruct((R, N), jnp.float32),
    mesh=_mesh,
    compiler_params=pltpu.CompilerParams(needs_layout_passes=False),
    scratch_types=[
        pltpu.VMEM((N,), jnp.float32),
        pltpu.VMEM((N,), jnp.float32),
        pltpu.VMEM((N,), jnp.float32),
        pltpu.VMEM((K,), jnp.float32),
        pltpu.SemaphoreType.DMA,
        pltpu.SemaphoreType.DMA,
        pltpu.SemaphoreType.DMA,
        pltpu.SemaphoreType.DMA,
        pltpu.SemaphoreType.DMA,
        pltpu.SemaphoreType.DMA,
    ],
)
def _sparsemax_sc(x_hbm, out_hbm, b0, b1, b2, cnt_v,
                  is0, is1, is2, os0, os1, os2):
    wid = lax.axis_index("s") * NC + lax.axis_index("c")
    bufs = (b0, b1, b2)
    isems = (is0, is1, is2)
    osems = (os0, os1, os2)

    zeros = jnp.zeros((L,), jnp.float32)
    ones = jnp.ones((L,), jnp.float32)
    lane_i = lax.iota(jnp.int32, L)
    lane_f = lane_i.astype(jnp.float32)
    inv_k = jnp.float32(1.0 / KS)

    @plsc.parallel_loop(0, WPL, unroll=4)
    def _(w):
        cnt_v[pl.ds(w * L, L)] = zeros

    row0 = wid * RPW
    in_h = {}
    out_h = {}
    for j in range(min(NBUF, RPW)):
        in_h[j] = pltpu.async_copy(x_hbm.at[row0 + j], bufs[j], isems[j])

    for j in range(RPW):
        b = j % NBUF
        row_v = bufs[b]
        if j >= NBUF - 1 and j + 1 < RPW:
            out_h[j + 1 - NBUF].wait()
            in_h[j + 1] = pltpu.async_copy(
                x_hbm.at[row0 + j + 1], bufs[(j + 1) % NBUF], isems[(j + 1) % NBUF]
            )
        in_h[j].wait()

        @plsc.parallel_loop(0, NV, ACC, unroll=2,
                            carry=tuple(jnp.full((L,), -jnp.inf, jnp.float32)
                                        for _ in range(ACC)))
        def m_accs(i, accs):
            return tuple(
                jnp.maximum(a, row_v[pl.ds((i + u) * L, L)])
                for u, a in enumerate(accs)
            )
        m_vec = functools.reduce(jnp.maximum, m_accs)
        m_s = jnp.max(m_vec)
        lo_s = m_s - jnp.float32(1.0)
        lo_v = jnp.full((L,), lo_s, jnp.float32)

        kf_v = jnp.full((L,), jnp.float32(KS), jnp.float32)

        @plsc.parallel_loop(0, NV, unroll=8)
        def _(i):
            x = row_v[pl.ds(i * L, L)]
            t = (x - lo_v) * kf_v
            idx = t.astype(jnp.int32)
            sidx = ((idx << 4) | (idx >> 7)) & (K - 1)
            plsc.addupdate_scatter(cnt_v, [sidx], ones, mask=x >= lo_v)

        lane_base = lo_v + lane_f * jnp.float32(WPL / KS)

        @plsc.parallel_loop(0, WPL, unroll=4, carry=(zeros, zeros))
        def tot(w, carry):
            c_run, s_run = carry
            v = cnt_v[pl.ds(w * L, L)]
            mid = lane_base + (w + 0.5) * inv_k
            return c_run + v, s_run + v * mid
        c_tot, s_tot = tot

        tc_excl = lax.rev(plsc.cumsum(lax.rev(c_tot, (0,))), (0,)) - c_tot
        ts_excl = lax.rev(plsc.cumsum(lax.rev(s_tot, (0,))), (0,)) - s_tot

        @plsc.parallel_loop(0, WPL, unroll=4,
                            carry=(tc_excl, ts_excl,
                                   jnp.full((L,), -3.0, jnp.float32)))
        def scan2(wr, carry):
            c_run, s_run, acc = carry
            w = WPL - 1 - wr
            v = cnt_v[pl.ds(w * L, L)]
            mid = lane_base + (w + 0.5) * inv_k
            cs = c_run + v
            ss = s_run + v * mid
            edge = lane_base + w * inv_k
            g = ss - edge * cs - jnp.float32(1.0)
            r = (ss - jnp.float32(1.0)) / jnp.maximum(cs, jnp.float32(1e-30))
            cnt_v[pl.ds(w * L, L)] = zeros
            return cs, ss, jnp.maximum(acc, jnp.where(g >= 0, r, jnp.float32(-3.0)))
        rmax = scan2[2]
        tau0 = jnp.clip(jnp.max(rmax), lo_s, m_s - jnp.float32(1.0 / N))
        tau0_v = jnp.full((L,), tau0, jnp.float32)

        @plsc.parallel_loop(0, NV, ACC, unroll=2,
                            carry=tuple(zeros for _ in range(2 * ACC)))
        def sk_accs(i, accs):
            s_out = []
            k_out = []
            for u in range(ACC):
                d = row_v[pl.ds((i + u) * L, L)] - tau0_v
                s_out.append(accs[u] + jnp.maximum(d, zeros))
                k_out.append(accs[ACC + u] + jnp.where(d > 0, ones, zeros))
            return tuple(s_out) + tuple(k_out)
        s_vec = functools.reduce(jnp.add, sk_accs[:ACC])
        k_vec = functools.reduce(jnp.add, sk_accs[ACC:])
        sp_v = jnp.full((L,), jnp.sum(s_vec), jnp.float32)
        kp_v = jnp.maximum(jnp.full((L,), jnp.sum(k_vec), jnp.float32), ones)
        tau1_v = tau0_v + (sp_v - ones) / kp_v

        @plsc.parallel_loop(0, NV, unroll=8)
        def _(i):
            sl = pl.ds(i * L, L)
            row_v[sl] = jnp.maximum(row_v[sl] - tau1_v, zeros)

        out_h[j] = pltpu.async_copy(row_v, out_hbm.at[row0 + j], osems[b])

    waited = {j + 1 - NBUF for j in range(RPW) if j >= NBUF - 1 and j + 1 < RPW}
    for j in range(RPW):
        if j not in waited:
            out_h[j].wait()


def kernel(input):
    return _sparsemax_sc(input)


# speedup vs baseline: 39.5603x; 1.1415x over previous
import functools

import jax
import jax.numpy as jnp
from jax import lax
from jax.experimental import pallas as pl
from jax.experimental.pallas import tpu as pltpu
from jax.experimental.pallas import tpu_sc as plsc

R = 128
N = 32768
L = 16
NV = N // L
K = 2048
KS = K * (1.0 - 2.0 ** -12)
WPL = K // L
NC = 2
NS = 16
NW = NC * NS
RPW = R // NW
ACC = 4
NBUF = 3

_mesh = plsc.VectorSubcoreMesh(
    core_axis_name="c", subcore_axis_name="s", num_cores=NC, num_subcores=NS
)


@functools.partial(
    pl.kernel,
    out_type=jax.ShapeDtypeStruct((R, N), jnp.float32),
    mesh=_mesh,
    compiler_params=pltpu.CompilerParams(needs_layout_passes=False),
    scratch_types=[
        pltpu.VMEM((N,), jnp.float32),
        pltpu.VMEM((N,), jnp.float32),
        pltpu.VMEM((N,), jnp.float32),
        pltpu.VMEM((K,), jnp.float32),
        pltpu.VMEM((K,), jnp.float32),
        pltpu.SemaphoreType.DMA,
        pltpu.SemaphoreType.DMA,
        pltpu.SemaphoreType.DMA,
        pltpu.SemaphoreType.DMA,
        pltpu.SemaphoreType.DMA,
        pltpu.SemaphoreType.DMA,
    ],
)
def _sparsemax_sc(x_hbm, out_hbm, b0, b1, b2, cnt_v, sum_v,
                  is0, is1, is2, os0, os1, os2):
    wid = lax.axis_index("s") * NC + lax.axis_index("c")
    bufs = (b0, b1, b2)
    isems = (is0, is1, is2)
    osems = (os0, os1, os2)

    zeros = jnp.zeros((L,), jnp.float32)
    ones = jnp.ones((L,), jnp.float32)
    lane_i = lax.iota(jnp.int32, L)
    lane_f = lane_i.astype(jnp.float32)
    inv_k = jnp.float32(1.0 / KS)

    @plsc.parallel_loop(0, WPL, unroll=4)
    def _(w):
        cnt_v[pl.ds(w * L, L)] = zeros
        sum_v[pl.ds(w * L, L)] = zeros

    row0 = wid * RPW
    in_h = {}
    out_h = {}
    for j in range(min(NBUF, RPW)):
        in_h[j] = pltpu.async_copy(x_hbm.at[row0 + j], bufs[j], isems[j])

    for j in range(RPW):
        b = j % NBUF
        row_v = bufs[b]
        if j >= NBUF - 1 and j + 1 < RPW:
            out_h[j + 1 - NBUF].wait()
            in_h[j + 1] = pltpu.async_copy(
                x_hbm.at[row0 + j + 1], bufs[(j + 1) % NBUF], isems[(j + 1) % NBUF]
            )
        in_h[j].wait()

        @plsc.parallel_loop(0, NV, ACC, unroll=2,
                            carry=tuple(jnp.full((L,), -jnp.inf, jnp.float32)
                                        for _ in range(ACC)))
        def m_accs(i, accs):
            return tuple(
                jnp.maximum(a, row_v[pl.ds((i + u) * L, L)])
                for u, a in enumerate(accs)
            )
        m_vec = functools.reduce(jnp.maximum, m_accs)
        m_s = jnp.max(m_vec)
        lo_s = m_s - jnp.float32(1.0)
        lo_v = jnp.full((L,), lo_s, jnp.float32)

        kf_v = jnp.full((L,), jnp.float32(KS), jnp.float32)

        @plsc.parallel_loop(0, NV, unroll=8)
        def _(i):
            x = row_v[pl.ds(i * L, L)]
            t = (x - lo_v) * kf_v
            idx = t.astype(jnp.int32)
            sidx = ((idx << 4) | (idx >> 7)) & (K - 1)
            m = x >= lo_v
            plsc.addupdate_scatter(cnt_v, [sidx], ones, mask=m)
            plsc.addupdate_scatter(sum_v, [sidx], x, mask=m)

        lane_base = lo_v + lane_f * jnp.float32(WPL / KS)

        @plsc.parallel_loop(0, WPL, unroll=4, carry=(zeros, zeros))
        def tot(w, carry):
            c_run, s_run = carry
            sl = pl.ds(w * L, L)
            return c_run + cnt_v[sl], s_run + sum_v[sl]
        c_tot, s_tot = tot

        tc_excl = lax.rev(plsc.cumsum(lax.rev(c_tot, (0,))), (0,)) - c_tot
        ts_excl = lax.rev(plsc.cumsum(lax.rev(s_tot, (0,))), (0,)) - s_tot

        @plsc.parallel_loop(0, WPL, unroll=4,
                            carry=(tc_excl, ts_excl,
                                   jnp.full((L,), -3.0, jnp.float32)))
        def scan2(wr, carry):
            c_run, s_run, acc = carry
            w = WPL - 1 - wr
            sl = pl.ds(w * L, L)
            cs = c_run + cnt_v[sl]
            ss = s_run + sum_v[sl]
            edge = lane_base + w * inv_k
            g = ss - edge * cs - jnp.float32(1.0)
            r = (ss - jnp.float32(1.0)) / jnp.maximum(cs, jnp.float32(1e-30))
            cnt_v[sl] = zeros
            sum_v[sl] = zeros
            return cs, ss, jnp.maximum(acc, jnp.where(g >= 0, r, jnp.float32(-3.0)))
        rmax = scan2[2]
        tau1_v = jnp.full(
            (L,), jnp.clip(jnp.max(rmax), lo_s, m_s - jnp.float32(1.0 / N)),
            jnp.float32)

        @plsc.parallel_loop(0, NV, unroll=8)
        def _(i):
            sl = pl.ds(i * L, L)
            row_v[sl] = jnp.maximum(row_v[sl] - tau1_v, zeros)

        out_h[j] = pltpu.async_copy(row_v, out_hbm.at[row0 + j], osems[b])

    waited = {j + 1 - NBUF for j in range(RPW) if j >= NBUF - 1 and j + 1 < RPW}
    for j in range(RPW):
        if j not in waited:
            out_h[j].wait()


def kernel(input):
    return _sparsemax_sc(input)


# speedup vs baseline: 39.6452x; 1.0021x over previous
import functools

import jax
import jax.numpy as jnp
from jax import lax
from jax.experimental import pallas as pl
from jax.experimental.pallas import tpu as pltpu
from jax.experimental.pallas import tpu_sc as plsc

R = 128
N = 32768
L = 16
NV = N // L
K = 2048
KS = K * (1.0 - 2.0 ** -12)
WPL = K // L
NC = 2
NS = 16
NW = NC * NS
RPW = R // NW
ACC = 4
NBUF = 3

_mesh = plsc.VectorSubcoreMesh(
    core_axis_name="c", subcore_axis_name="s", num_cores=NC, num_subcores=NS
)


@functools.partial(
    pl.kernel,
    out_type=jax.ShapeDtypeStruct((R, N), jnp.float32),
    mesh=_mesh,
    compiler_params=pltpu.CompilerParams(needs_layout_passes=False),
    scratch_types=[
        pltpu.VMEM((N,), jnp.float32),
        pltpu.VMEM((N,), jnp.float32),
        pltpu.VMEM((N,), jnp.float32),
        pltpu.VMEM((K,), jnp.float32),
        pltpu.VMEM((K,), jnp.float32),
        pltpu.SemaphoreType.DMA,
        pltpu.SemaphoreType.DMA,
        pltpu.SemaphoreType.DMA,
        pltpu.SemaphoreType.DMA,
        pltpu.SemaphoreType.DMA,
        pltpu.SemaphoreType.DMA,
    ],
)
def _sparsemax_sc(x_hbm, out_hbm, b0, b1, b2, cnt_v, sum_v,
                  is0, is1, is2, os0, os1, os2):
    wid = lax.axis_index("s") * NC + lax.axis_index("c")
    bufs = (b0, b1, b2)
    isems = (is0, is1, is2)
    osems = (os0, os1, os2)

    zeros = jnp.zeros((L,), jnp.float32)
    ones = jnp.ones((L,), jnp.float32)
    lane_i = lax.iota(jnp.int32, L)
    lane_f = lane_i.astype(jnp.float32)
    inv_k = jnp.float32(1.0 / KS)

    @plsc.parallel_loop(0, WPL, unroll=4)
    def _(w):
        cnt_v[pl.ds(w * L, L)] = zeros
        sum_v[pl.ds(w * L, L)] = zeros

    row0 = wid * RPW
    in_h = {}
    out_h = {}
    for j in range(min(NBUF, RPW)):
        in_h[j] = pltpu.async_copy(x_hbm.at[row0 + j], bufs[j], isems[j])

    in_h[0].wait()

    @plsc.parallel_loop(0, NV, ACC, unroll=2,
                        carry=tuple(jnp.full((L,), -jnp.inf, jnp.float32)
                                    for _ in range(ACC)))
    def m_accs0(i, accs):
        return tuple(
            jnp.maximum(a, b0[pl.ds((i + u) * L, L)])
            for u, a in enumerate(accs)
        )
    m_vec = functools.reduce(jnp.maximum, m_accs0)

    for j in range(RPW):
        b = j % NBUF
        row_v = bufs[b]
        if j >= NBUF - 1 and j + 1 < RPW:
            out_h[j + 1 - NBUF].wait()
            in_h[j + 1] = pltpu.async_copy(
                x_hbm.at[row0 + j + 1], bufs[(j + 1) % NBUF], isems[(j + 1) % NBUF]
            )

        m_s = jnp.max(m_vec)
        lo_s = m_s - jnp.float32(1.0)
        lo_v = jnp.full((L,), lo_s, jnp.float32)

        kf_v = jnp.full((L,), jnp.float32(KS), jnp.float32)

        @plsc.parallel_loop(0, NV, unroll=8)
        def _(i):
            x = row_v[pl.ds(i * L, L)]
            t = (x - lo_v) * kf_v
            idx = t.astype(jnp.int32)
            sidx = ((idx << 4) | (idx >> 7)) & (K - 1)
            m = x >= lo_v
            plsc.addupdate_scatter(cnt_v, [sidx], ones, mask=m)
            plsc.addupdate_scatter(sum_v, [sidx], x, mask=m)

        lane_base = lo_v + lane_f * jnp.float32(WPL / KS)

        @plsc.parallel_loop(0, WPL, unroll=4, carry=(zeros, zeros))
        def tot(w, carry):
            c_run, s_run = carry
            sl = pl.ds(w * L, L)
            return c_run + cnt_v[sl], s_run + sum_v[sl]
        c_tot, s_tot = tot

        tc_excl = lax.rev(plsc.cumsum(lax.rev(c_tot, (0,))), (0,)) - c_tot
        ts_excl = lax.rev(plsc.cumsum(lax.rev(s_tot, (0,))), (0,)) - s_tot

        @plsc.parallel_loop(0, WPL, unroll=4,
                            carry=(tc_excl, ts_excl,
                                   jnp.full((L,), -3.0, jnp.float32)))
        def scan2(wr, carry):
            c_run, s_run, acc = carry
            w = WPL - 1 - wr
            sl = pl.ds(w * L, L)
            cs = c_run + cnt_v[sl]
            ss = s_run + sum_v[sl]
            edge = lane_base + w * inv_k
            g = ss - edge * cs - jnp.float32(1.0)
            r = (ss - jnp.float32(1.0)) / jnp.maximum(cs, jnp.float32(1e-30))
            cnt_v[sl] = zeros
            sum_v[sl] = zeros
            return cs, ss, jnp.maximum(acc, jnp.where(g >= 0, r, jnp.float32(-3.0)))
        rmax = scan2[2]
        tau1_v = jnp.full(
            (L,), jnp.clip(jnp.max(rmax), lo_s, m_s - jnp.float32(1.0 / N)),
            jnp.float32)

        if j + 1 < RPW:
            nxt_v = bufs[(j + 1) % NBUF]
            in_h[j + 1].wait()

            @plsc.parallel_loop(0, NV, ACC, unroll=2,
                                carry=tuple(jnp.full((L,), -jnp.inf, jnp.float32)
                                            for _ in range(ACC)))
            def cm_accs(i, accs):
                outs = []
                for u, a in enumerate(accs):
                    sl = pl.ds((i + u) * L, L)
                    row_v[sl] = jnp.maximum(row_v[sl] - tau1_v, zeros)
                    outs.append(jnp.maximum(a, nxt_v[sl]))
                return tuple(outs)
            m_vec = functools.reduce(jnp.maximum, cm_accs)
        else:
            @plsc.parallel_loop(0, NV, unroll=8)
            def _(i):
                sl = pl.ds(i * L, L)
                row_v[sl] = jnp.maximum(row_v[sl] - tau1_v, zeros)

        out_h[j] = pltpu.async_copy(row_v, out_hbm.at[row0 + j], osems[b])

    waited = {j + 1 - NBUF for j in range(RPW) if j >= NBUF - 1 and j + 1 < RPW}
    for j in range(RPW):
        if j not in waited:
            out_h[j].wait()


def kernel(input):
    return _sparsemax_sc(input)


# speedup vs baseline: 40.0643x; 1.0106x over previous
import functools

import jax
import jax.numpy as jnp
from jax import lax
from jax.experimental import pallas as pl
from jax.experimental.pallas import tpu as pltpu
from jax.experimental.pallas import tpu_sc as plsc

R = 128
N = 32768
L = 16
NV = N // L
K = 1024
KS = K * (1.0 - 2.0 ** -12)
WPL = K // L
NC = 2
NS = 16
NW = NC * NS
RPW = R // NW
ACC = 4
NBUF = 3
SHW = 6

_mesh = plsc.VectorSubcoreMesh(
    core_axis_name="c", subcore_axis_name="s", num_cores=NC, num_subcores=NS
)


@functools.partial(
    pl.kernel,
    out_type=jax.ShapeDtypeStruct((R, N), jnp.float32),
    mesh=_mesh,
    compiler_params=pltpu.CompilerParams(needs_layout_passes=False),
    scratch_types=[
        pltpu.VMEM((N,), jnp.float32),
        pltpu.VMEM((N,), jnp.float32),
        pltpu.VMEM((N,), jnp.float32),
        pltpu.VMEM((K,), jnp.float32),
        pltpu.VMEM((K,), jnp.float32),
        pltpu.SemaphoreType.DMA,
        pltpu.SemaphoreType.DMA,
        pltpu.SemaphoreType.DMA,
        pltpu.SemaphoreType.DMA,
        pltpu.SemaphoreType.DMA,
        pltpu.SemaphoreType.DMA,
    ],
)
def _sparsemax_sc(x_hbm, out_hbm, b0, b1, b2, cnt_v, sum_v,
                  is0, is1, is2, os0, os1, os2):
    wid = lax.axis_index("s") * NC + lax.axis_index("c")
    bufs = (b0, b1, b2)
    isems = (is0, is1, is2)
    osems = (os0, os1, os2)

    zeros = jnp.zeros((L,), jnp.float32)
    ones = jnp.ones((L,), jnp.float32)
    lane_i = lax.iota(jnp.int32, L)
    lane_f = lane_i.astype(jnp.float32)
    inv_k = jnp.float32(1.0 / KS)

    @plsc.parallel_loop(0, WPL, unroll=4)
    def _(w):
        cnt_v[pl.ds(w * L, L)] = zeros
        sum_v[pl.ds(w * L, L)] = zeros

    row0 = wid * RPW
    in_h = {}
    out_h = {}
    for j in range(min(NBUF, RPW)):
        in_h[j] = pltpu.async_copy(x_hbm.at[row0 + j], bufs[j], isems[j])

    in_h[0].wait()

    @plsc.parallel_loop(0, NV, ACC, unroll=2,
                        carry=tuple(jnp.full((L,), -jnp.inf, jnp.float32)
                                    for _ in range(ACC)))
    def m_accs0(i, accs):
        return tuple(
            jnp.maximum(a, b0[pl.ds((i + u) * L, L)])
            for u, a in enumerate(accs)
        )
    m_vec = functools.reduce(jnp.maximum, m_accs0)

    for j in range(RPW):
        b = j % NBUF
        row_v = bufs[b]
        if j >= NBUF - 1 and j + 1 < RPW:
            out_h[j + 1 - NBUF].wait()
            in_h[j + 1] = pltpu.async_copy(
                x_hbm.at[row0 + j + 1], bufs[(j + 1) % NBUF], isems[(j + 1) % NBUF]
            )

        m_s = jnp.max(m_vec)
        lo_s = m_s - jnp.float32(1.0)
        lo_v = jnp.full((L,), lo_s, jnp.float32)

        kf_v = jnp.full((L,), jnp.float32(KS), jnp.float32)

        @plsc.parallel_loop(0, NV, unroll=8)
        def _(i):
            x = row_v[pl.ds(i * L, L)]
            t = (x - lo_v) * kf_v
            idx = t.astype(jnp.int32)
            sidx = ((idx << 4) | (idx >> SHW)) & (K - 1)
            m = x >= lo_v
            plsc.addupdate_scatter(cnt_v, [sidx], ones, mask=m)
            plsc.addupdate_scatter(sum_v, [sidx], x, mask=m)

        lane_base = lo_v + lane_f * jnp.float32(WPL / KS)

        @plsc.parallel_loop(0, WPL, unroll=8, carry=(zeros, zeros))
        def tot(w, carry):
            c_run, s_run = carry
            sl = pl.ds(w * L, L)
            return c_run + cnt_v[sl], s_run + sum_v[sl]
        c_tot, s_tot = tot

        tc_excl = lax.rev(plsc.cumsum(lax.rev(c_tot, (0,))), (0,)) - c_tot
        ts_excl = lax.rev(plsc.cumsum(lax.rev(s_tot, (0,))), (0,)) - s_tot

        @plsc.parallel_loop(0, WPL, unroll=8,
                            carry=(tc_excl, ts_excl,
                                   jnp.full((L,), -3.0, jnp.float32)))
        def scan2(wr, carry):
            c_run, s_run, acc = carry
            w = WPL - 1 - wr
            sl = pl.ds(w * L, L)
            cs = c_run + cnt_v[sl]
            ss = s_run + sum_v[sl]
            edge = lane_base + w * inv_k
            g = ss - edge * cs - jnp.float32(1.0)
            r = (ss - jnp.float32(1.0)) / jnp.maximum(cs, jnp.float32(1e-30))
            cnt_v[sl] = zeros
            sum_v[sl] = zeros
            return cs, ss, jnp.maximum(acc, jnp.where(g >= 0, r, jnp.float32(-3.0)))
        rmax = scan2[2]
        tau1_v = jnp.full(
            (L,), jnp.clip(jnp.max(rmax), lo_s, m_s - jnp.float32(1.0 / N)),
            jnp.float32)

        if j + 1 < RPW:
            nxt_v = bufs[(j + 1) % NBUF]
            in_h[j + 1].wait()

            @plsc.parallel_loop(0, NV, ACC, unroll=2,
                                carry=tuple(jnp.full((L,), -jnp.inf, jnp.float32)
                                            for _ in range(ACC)))
            def cm_accs(i, accs):
                outs = []
                for u, a in enumerate(accs):
                    sl = pl.ds((i + u) * L, L)
                    row_v[sl] = jnp.maximum(row_v[sl] - tau1_v, zeros)
                    outs.append(jnp.maximum(a, nxt_v[sl]))
                return tuple(outs)
            m_vec = functools.reduce(jnp.maximum, cm_accs)
        else:
            @plsc.parallel_loop(0, NV, unroll=8)
            def _(i):
                sl = pl.ds(i * L, L)
                row_v[sl] = jnp.maximum(row_v[sl] - tau1_v, zeros)

        out_h[j] = pltpu.async_copy(row_v, out_hbm.at[row0 + j], osems[b])

    waited = {j + 1 - NBUF for j in range(RPW) if j >= NBUF - 1 and j + 1 < RPW}
    for j in range(RPW):
        if j not in waited:
            out_h[j].wait()


def kernel(input):
    return _sparsemax_sc(input)
